# baseline (speedup 1.0000x reference)
import dataclasses
import functools

import jax
import jax.numpy as jnp
from jax import lax
from jax.experimental import pallas as pl
from jax.experimental.pallas import tpu as pltpu
from jax.experimental.pallas import tpu_sc as plsc

F32 = jnp.float32
NC = 2
NS = 16
NW = NC * NS
LANES = 16

AGG_WIN = 128
SLOP = 8

_MESH = dict(core_axis_name="c", subcore_axis_name="s", num_cores=NC,
             num_subcores=NS)

_SC_PARAMS = pltpu.CompilerParams()
if "needs_layout_passes" in pltpu.CompilerParams.__dataclass_fields__:
    _SC_PARAMS = dataclasses.replace(_SC_PARAMS, needs_layout_passes=False)


def _sc_degree(dst_rows, n_nodes):
    per = dst_rows.shape[1]

    @functools.partial(
        pl.kernel,
        out_type=jax.ShapeDtypeStruct((NW, n_nodes), F32),
        mesh=plsc.VectorSubcoreMesh(**_MESH),
        scratch_types=[
            pltpu.VMEM((per,), jnp.int32),
            pltpu.VMEM((n_nodes,), F32),
            pltpu.SemaphoreType.DMA,
        ],
        compiler_params=_SC_PARAMS,
    )
    def k(dst_hbm, o_hbm, idx_v, acc_v, sem):
        c = lax.axis_index("c")
        s = lax.axis_index("s")
        wid = s * NC + c
        pltpu.async_copy(dst_hbm.at[wid], idx_v, sem).wait()
        zeros = jnp.zeros((LANES,), F32)
        ones = jnp.ones((LANES,), F32)

        @pl.loop(0, n_nodes, step=LANES)
        def _(i):
            acc_v[pl.ds(i, LANES)] = zeros

        @pl.loop(0, per, step=LANES)
        def _(i):
            idx = idx_v[pl.ds(i, LANES)]
            plsc.addupdate_scatter(acc_v, [idx], ones)

        pltpu.async_copy(acc_v, o_hbm.at[wid], sem).wait()

    return k(dst_rows)


def _sc_aggregate(table, srcw, dstw, zeros):
    n, h = table.shape
    nwin, win = srcw.shape

    @functools.partial(
        pl.kernel,
        out_type=jax.ShapeDtypeStruct((NC, n, h), F32),
        mesh=plsc.VectorSubcoreMesh(**_MESH),
        scratch_types=[
            pltpu.VMEM_SHARED((n + SLOP, h), F32),
            pltpu.VMEM((2, win, h), F32),
            pltpu.VMEM((win,), jnp.int32),
            pltpu.SemaphoreType.DMA,
            pltpu.SemaphoreType.DMA,
        ],
    )
    def k(t_hbm, s_hbm, d_hbm, z_hbm, o_hbm, acc_sh, rows_v, pidx_v, sem,
          sem_a):
        c = lax.axis_index("c")
        s = lax.axis_index("s")

        @pl.when(s == 0)
        def _():
            pltpu.async_copy(z_hbm, acc_sh, sem).wait()

        nvec = jnp.full((LANES,), n, jnp.int32)

        @pl.loop(0, win, step=LANES)
        def _(i):
            pidx_v[pl.ds(i, LANES)] = nvec

        pltpu.async_copy(rows_v.at[0], acc_sh.at[pidx_v], sem_a, add=True)

        plsc.subcore_barrier()

        def drain():
            pltpu.make_async_copy(
                t_hbm.at[pl.ds(0, win)], rows_v.at[0], sem_a).wait()

        def body(si_v, di_v):
            drain()
            pltpu.sync_copy(t_hbm.at[si_v.at[0]], rows_v.at[0])
            pltpu.async_copy(rows_v.at[0], acc_sh.at[di_v.at[0]], sem_a,
                             add=True)
            pltpu.sync_copy(t_hbm.at[si_v.at[1]], rows_v.at[1])
            drain()
            pltpu.async_copy(rows_v.at[1], acc_sh.at[di_v.at[1]], sem_a,
                             add=True)

        pltpu.emit_pipeline(
            body,
            grid=(nwin // 2,),
            in_specs=[
                pl.BlockSpec((2, win), lambda i: (i, 0)),
                pl.BlockSpec((2, win), lambda i: (i, 0)),
            ],
            out_specs=[],
            core_axis_name=("c", "s"),
            dimension_semantics=(pltpu.PARALLEL,),
        )(s_hbm, d_hbm)

        drain()
        plsc.subcore_barrier()

        @pl.when(s == 0)
        def _():
            pltpu.async_copy(acc_sh.at[pl.ds(0, n)], o_hbm.at[c], sem).wait()

    return k(table, srcw, dstw, zeros)


def _sc_edge_gather(a, b, src2d, dst2d):
    n, h = a.shape
    e = src2d.shape[1]
    win = 128

    @functools.partial(
        pl.kernel,
        out_type=[
            jax.ShapeDtypeStruct((e, h), F32),
            jax.ShapeDtypeStruct((e, h), F32),
        ],
        mesh=plsc.VectorSubcoreMesh(**_MESH),
        scratch_types=[
            pltpu.SemaphoreType.DMA,
            pltpu.SemaphoreType.DMA,
        ],
    )
    def k(a_hbm, b_hbm, s_hbm, d_hbm, ga_hbm, gb_hbm, sem1, sem2):
        def body(si_v, di_v, ga_v, gb_v):
            c1 = pltpu.async_copy(a_hbm.at[si_v.at[0]], ga_v, sem1)
            c2 = pltpu.async_copy(b_hbm.at[di_v.at[0]], gb_v, sem2)
            c1.wait()
            c2.wait()

        pltpu.emit_pipeline(
            body,
            grid=(e // win,),
            in_specs=[
                pl.BlockSpec((1, win), lambda i: (0, i)),
                pl.BlockSpec((1, win), lambda i: (0, i)),
            ],
            out_specs=[
                pl.BlockSpec((win, h), lambda i: (i, 0)),
                pl.BlockSpec((win, h), lambda i: (i, 0)),
            ],
            core_axis_name=("c", "s"),
            dimension_semantics=(pltpu.PARALLEL,),
        )(s_hbm, d_hbm, ga_hbm, gb_hbm)

    return k(a, b, src2d, dst2d)


def _tc_prep(degp, x, w0):
    n, d = x.shape
    h = w0.shape[1]

    def body(degp_ref, x_ref, w_ref, dis_ref, t_ref):
        deg = jnp.sum(degp_ref[...], axis=0, keepdims=True) + 1.0
        dis_col = jnp.reshape(lax.rsqrt(deg), (n, 1))
        xx = x_ref[...]
        xx = jnp.where(jnp.isnan(xx), 0.0, xx)
        t = jnp.dot(xx, w_ref[...], preferred_element_type=F32)
        dis_ref[...] = dis_col
        t_ref[...] = t * dis_col

    return pl.pallas_call(
        body,
        out_shape=[
            jax.ShapeDtypeStruct((n, 1), F32),
            jax.ShapeDtypeStruct((n, h), F32),
        ],
    )(degp, x, w0)


def _tc_post(parts, table, dis, bias, g, be, wnext, scale_next):
    n, h = table.shape
    outs = [jax.ShapeDtypeStruct((n, w.shape[1]), F32) for w in wnext]

    def body(p_ref, t_ref, dis_ref, b_ref, g_ref, be_ref, *w_and_out):
        w_refs = w_and_out[:len(wnext)]
        o_refs = w_and_out[len(wnext):]
        dis_col = dis_ref[...]
        acc = p_ref[0] + p_ref[1] + t_ref[...]
        hh = acc * dis_col + b_ref[...]
        m = jnp.mean(hh, axis=0, keepdims=True)
        cen = hh - m
        v = jnp.mean(cen * cen, axis=0, keepdims=True)
        hh = cen * lax.rsqrt(v + 1e-5) * g_ref[...] + be_ref[...]
        hh = jnp.maximum(hh, 0.0)
        for w_ref, o_ref in zip(w_refs, o_refs):
            t = jnp.dot(hh, w_ref[...], preferred_element_type=F32)
            if scale_next:
                t = t * dis_col
            o_ref[...] = t

    return pl.pallas_call(body, out_shape=outs)(
        parts, table, dis, bias, g, be, *wnext)


def _tc_classifier(ga, gb, eat, w1e, cb1, cw2, cb2, cw3, cb3):
    e, h = ga.shape
    de = eat.shape[0]
    h2 = cw2.shape[1]
    out_dim = cw3.shape[1]
    tile = 6400

    def body(ga_ref, gb_ref, ea_ref, w1e_ref, cb1_ref, w2_ref, cb2_ref,
             w3_ref, cb3_ref, o_ref):
        eav = ea_ref[...]
        eav = jnp.where(jnp.isnan(eav), 0.0, eav)
        z = ga_ref[...] + gb_ref[...]
        z = z + lax.dot_general(eav, w1e_ref[...], (((0,), (0,)), ((), ())),
                                preferred_element_type=F32)
        z = jnp.maximum(z + cb1_ref[...], 0.0)
        z = jnp.dot(z, w2_ref[...], preferred_element_type=F32) + cb2_ref[...]
        z = jnp.maximum(z, 0.0)
        zt = lax.dot_general(w3_ref[...], z, (((0,), (1,)), ((), ())),
                             preferred_element_type=F32)
        o_ref[...] = zt + cb3_ref[...]

    full = lambda shape: pl.BlockSpec(shape, lambda i: (0, 0))
    return pl.pallas_call(
        body,
        grid=(e // tile,),
        in_specs=[
            pl.BlockSpec((tile, h), lambda i: (i, 0)),
            pl.BlockSpec((tile, h), lambda i: (i, 0)),
            pl.BlockSpec((de, tile), lambda i: (0, i)),
            full((de, h)),
            full((1, h)),
            full((h, h2)),
            full((1, h2)),
            full((h2, out_dim)),
            full((out_dim, 1)),
        ],
        out_specs=pl.BlockSpec((out_dim, tile), lambda i: (0, i)),
        out_shape=jax.ShapeDtypeStruct((out_dim, e), F32),
    )(ga, gb, eat, w1e, cb1, cw2, cb2, cw3, cb3)


def kernel(x, edge_index, edge_attr, W0, b0, W1, b1, W2, b2, g0, be0, g1,
           be1, g2, be2, cW1, cb1, cW2, cb2, cW3, cb3):
    n, d = x.shape
    e = edge_index.shape[1]
    h = W0.shape[1]

    src = edge_index[0]
    dst = edge_index[1]
    src2d = src.reshape(1, e)
    dst2d = dst.reshape(1, e)

    srcw = src.reshape(e // AGG_WIN, AGG_WIN)
    dstw = dst.reshape(e // AGG_WIN, AGG_WIN)
    zeros_nh = jnp.zeros((n + SLOP, h), F32)
    row = lambda v: v.reshape(1, -1)

    degp = _sc_degree(dst.reshape(NW, e // NW), n)
    dis, table = _tc_prep(degp, x, W0)

    parts = _sc_aggregate(table, srcw, dstw, zeros_nh)
    (table,) = _tc_post(parts, table, dis, row(b0), row(g0), row(be0),
                        [W1], True)
    parts = _sc_aggregate(table, srcw, dstw, zeros_nh)
    (table,) = _tc_post(parts, table, dis, row(b1), row(g1), row(be1),
                        [W2], True)
    parts = _sc_aggregate(table, srcw, dstw, zeros_nh)
    a, b = _tc_post(parts, table, dis, row(b2), row(g2), row(be2),
                    [cW1[:h], cW1[h:2 * h]], False)

    eat = edge_attr.T
    eh = e // 2
    outs = []
    for p in range(2):
        sl = slice(p * eh, (p + 1) * eh)
        ga, gb = _sc_edge_gather(a, b, src2d[:, sl], dst2d[:, sl])
        outs.append(_tc_classifier(ga, gb, eat[:, sl], cW1[2 * h:],
                                   row(cb1), cW2, row(cb2), cW3,
                                   cb3.reshape(-1, 1)))
    return jnp.concatenate(outs, axis=1).T


# speedup vs baseline: 1.0126x; 1.0126x over previous
import dataclasses
import functools

import jax
import jax.numpy as jnp
from jax import lax
from jax.experimental import pallas as pl
from jax.experimental.pallas import tpu as pltpu
from jax.experimental.pallas import tpu_sc as plsc

F32 = jnp.float32
NC = 2
NS = 16
NW = NC * NS
LANES = 16

AGG_WIN = 128
SLOP = 8

_MESH = dict(core_axis_name="c", subcore_axis_name="s", num_cores=NC,
             num_subcores=NS)

_SC_PARAMS = pltpu.CompilerParams()
if "needs_layout_passes" in pltpu.CompilerParams.__dataclass_fields__:
    _SC_PARAMS = dataclasses.replace(_SC_PARAMS, needs_layout_passes=False)


def _sc_degree(dst_rows, n_nodes):
    per = dst_rows.shape[1]

    @functools.partial(
        pl.kernel,
        out_type=jax.ShapeDtypeStruct((NW, n_nodes), F32),
        mesh=plsc.VectorSubcoreMesh(**_MESH),
        scratch_types=[
            pltpu.VMEM((per,), jnp.int32),
            pltpu.VMEM((n_nodes,), F32),
            pltpu.SemaphoreType.DMA,
        ],
        compiler_params=_SC_PARAMS,
    )
    def k(dst_hbm, o_hbm, idx_v, acc_v, sem):
        c = lax.axis_index("c")
        s = lax.axis_index("s")
        wid = s * NC + c
        pltpu.async_copy(dst_hbm.at[wid], idx_v, sem).wait()
        zeros = jnp.zeros((LANES,), F32)
        ones = jnp.ones((LANES,), F32)

        @pl.loop(0, n_nodes, step=LANES)
        def _(i):
            acc_v[pl.ds(i, LANES)] = zeros

        @pl.loop(0, per, step=LANES)
        def _(i):
            idx = idx_v[pl.ds(i, LANES)]
            plsc.addupdate_scatter(acc_v, [idx], ones)

        pltpu.async_copy(acc_v, o_hbm.at[wid], sem).wait()

    return k(dst_rows)


def _sc_aggregate(table, srcw, dstw, zeros):
    n, h = table.shape
    nwin, win = srcw.shape

    @functools.partial(
        pl.kernel,
        out_type=jax.ShapeDtypeStruct((NC, n, h), F32),
        mesh=plsc.VectorSubcoreMesh(**_MESH),
        scratch_types=[
            pltpu.VMEM_SHARED((n + SLOP, h), F32),
            pltpu.VMEM((2, win, h), F32),
            pltpu.VMEM((win,), jnp.int32),
            pltpu.SemaphoreType.DMA,
            pltpu.SemaphoreType.DMA,
        ],
    )
    def k(t_hbm, s_hbm, d_hbm, z_hbm, o_hbm, acc_sh, rows_v, pidx_v, sem,
          sem_a):
        c = lax.axis_index("c")
        s = lax.axis_index("s")

        @pl.when(s == 0)
        def _():
            pltpu.async_copy(z_hbm, acc_sh, sem).wait()

        nvec = jnp.full((LANES,), n, jnp.int32)

        @pl.loop(0, win, step=LANES)
        def _(i):
            pidx_v[pl.ds(i, LANES)] = nvec

        pltpu.async_copy(rows_v.at[0], acc_sh.at[pidx_v], sem_a, add=True)

        plsc.subcore_barrier()

        def drain():
            pltpu.make_async_copy(
                t_hbm.at[pl.ds(0, win)], rows_v.at[0], sem_a).wait()

        def body(si_v, di_v):
            drain()
            pltpu.sync_copy(t_hbm.at[si_v.at[0]], rows_v.at[0])
            pltpu.async_copy(rows_v.at[0], acc_sh.at[di_v.at[0]], sem_a,
                             add=True)
            pltpu.sync_copy(t_hbm.at[si_v.at[1]], rows_v.at[1])
            drain()
            pltpu.async_copy(rows_v.at[1], acc_sh.at[di_v.at[1]], sem_a,
                             add=True)

        pltpu.emit_pipeline(
            body,
            grid=(nwin // 2,),
            in_specs=[
                pl.BlockSpec((2, win), lambda i: (i, 0)),
                pl.BlockSpec((2, win), lambda i: (i, 0)),
            ],
            out_specs=[],
            core_axis_name=("c", "s"),
            dimension_semantics=(pltpu.PARALLEL,),
        )(s_hbm, d_hbm)

        drain()
        plsc.subcore_barrier()

        @pl.when(s == 0)
        def _():
            pltpu.async_copy(acc_sh.at[pl.ds(0, n)], o_hbm.at[c], sem).wait()

    return k(table, srcw, dstw, zeros)


def _sc_edge_gather(a, b, src2d, dst2d):
    n, h = a.shape
    e = src2d.shape[1]
    win = 128

    @functools.partial(
        pl.kernel,
        out_type=[
            jax.ShapeDtypeStruct((e, h), F32),
            jax.ShapeDtypeStruct((e, h), F32),
        ],
        mesh=plsc.VectorSubcoreMesh(**_MESH),
        scratch_types=[
            pltpu.SemaphoreType.DMA,
            pltpu.SemaphoreType.DMA,
        ],
    )
    def k(a_hbm, b_hbm, s_hbm, d_hbm, ga_hbm, gb_hbm, sem1, sem2):
        def body(si_v, di_v, ga_v, gb_v):
            c1 = pltpu.async_copy(a_hbm.at[si_v.at[0]], ga_v, sem1)
            c2 = pltpu.async_copy(b_hbm.at[di_v.at[0]], gb_v, sem2)
            c1.wait()
            c2.wait()

        pltpu.emit_pipeline(
            body,
            grid=(e // win,),
            in_specs=[
                pl.BlockSpec((1, win), lambda i: (0, i)),
                pl.BlockSpec((1, win), lambda i: (0, i)),
            ],
            out_specs=[
                pl.BlockSpec((win, h), lambda i: (i, 0)),
                pl.BlockSpec((win, h), lambda i: (i, 0)),
            ],
            core_axis_name=("c", "s"),
            dimension_semantics=(pltpu.PARALLEL,),
        )(s_hbm, d_hbm, ga_hbm, gb_hbm)

    return k(a, b, src2d, dst2d)


def _tc_prep(degp, x, w0):
    n, d = x.shape
    h = w0.shape[1]

    def body(degp_ref, x_ref, w_ref, dis_ref, t_ref):
        deg = jnp.sum(degp_ref[...], axis=0, keepdims=True) + 1.0
        dis_col = jnp.reshape(lax.rsqrt(deg), (n, 1))
        xx = x_ref[...]
        xx = jnp.where(jnp.isnan(xx), 0.0, xx)
        t = jnp.dot(xx, w_ref[...], preferred_element_type=F32)
        dis_ref[...] = dis_col
        t_ref[...] = t * dis_col

    return pl.pallas_call(
        body,
        out_shape=[
            jax.ShapeDtypeStruct((n, 1), F32),
            jax.ShapeDtypeStruct((n, h), F32),
        ],
    )(degp, x, w0)


def _tc_post(parts, table, dis, bias, g, be, wnext, scale_next):
    n, h = table.shape
    outs = [jax.ShapeDtypeStruct((n, w.shape[1]), F32) for w in wnext]

    def body(p_ref, t_ref, dis_ref, b_ref, g_ref, be_ref, *w_and_out):
        w_refs = w_and_out[:len(wnext)]
        o_refs = w_and_out[len(wnext):]
        dis_col = dis_ref[...]
        acc = p_ref[0] + p_ref[1] + t_ref[...]
        hh = acc * dis_col + b_ref[...]
        m = jnp.mean(hh, axis=0, keepdims=True)
        cen = hh - m
        v = jnp.mean(cen * cen, axis=0, keepdims=True)
        hh = cen * lax.rsqrt(v + 1e-5) * g_ref[...] + be_ref[...]
        hh = jnp.maximum(hh, 0.0)
        for w_ref, o_ref in zip(w_refs, o_refs):
            t = jnp.dot(hh, w_ref[...], preferred_element_type=F32)
            if scale_next:
                t = t * dis_col
            o_ref[...] = t

    return pl.pallas_call(body, out_shape=outs)(
        parts, table, dis, bias, g, be, *wnext)


def _tc_classifier(ga, gb, eat, w1e, cb1, cw2, cb2, cw3, cb3):
    e, h = ga.shape
    de = eat.shape[0]
    h2 = cw2.shape[1]
    out_dim = cw3.shape[1]
    tile = 6400

    def body(ga_ref, gb_ref, ea_ref, w1e_ref, cb1_ref, w2_ref, cb2_ref,
             w3_ref, cb3_ref, o_ref):
        eav = ea_ref[...]
        eav = jnp.where(jnp.isnan(eav), 0.0, eav)
        z = ga_ref[...] + gb_ref[...]
        z = z + lax.dot_general(eav, w1e_ref[...], (((0,), (0,)), ((), ())),
                                preferred_element_type=F32)
        z = jnp.maximum(z + cb1_ref[...], 0.0)
        z = jnp.dot(z, w2_ref[...], preferred_element_type=F32) + cb2_ref[...]
        z = jnp.maximum(z, 0.0)
        zt = lax.dot_general(w3_ref[...], z, (((0,), (1,)), ((), ())),
                             preferred_element_type=F32)
        o_ref[...] = zt + cb3_ref[...]

    full = lambda shape: pl.BlockSpec(shape, lambda i: (0, 0))
    return pl.pallas_call(
        body,
        grid=(e // tile,),
        in_specs=[
            pl.BlockSpec((tile, h), lambda i: (i, 0)),
            pl.BlockSpec((tile, h), lambda i: (i, 0)),
            pl.BlockSpec((de, tile), lambda i: (0, i)),
            full((de, h)),
            full((1, h)),
            full((h, h2)),
            full((1, h2)),
            full((h2, out_dim)),
            full((out_dim, 1)),
        ],
        out_specs=pl.BlockSpec((out_dim, tile), lambda i: (0, i)),
        out_shape=jax.ShapeDtypeStruct((out_dim, e), F32),
    )(ga, gb, eat, w1e, cb1, cw2, cb2, cw3, cb3)


def kernel(x, edge_index, edge_attr, W0, b0, W1, b1, W2, b2, g0, be0, g1,
           be1, g2, be2, cW1, cb1, cW2, cb2, cW3, cb3):
    n, d = x.shape
    e = edge_index.shape[1]
    h = W0.shape[1]

    src = edge_index[0]
    dst = edge_index[1]
    src2d = src.reshape(1, e)
    dst2d = dst.reshape(1, e)

    srcw = src.reshape(e // AGG_WIN, AGG_WIN)
    dstw = dst.reshape(e // AGG_WIN, AGG_WIN)
    zeros_nh = jnp.zeros((n + SLOP, h), F32)
    row = lambda v: v.reshape(1, -1)

    degp = _sc_degree(dst.reshape(NW, e // NW), n)
    dis, table = _tc_prep(degp, x, W0)

    parts = _sc_aggregate(table, srcw, dstw, zeros_nh)
    (table,) = _tc_post(parts, table, dis, row(b0), row(g0), row(be0),
                        [W1], True)
    parts = _sc_aggregate(table, srcw, dstw, zeros_nh)
    (table,) = _tc_post(parts, table, dis, row(b1), row(g1), row(be1),
                        [W2], True)
    parts = _sc_aggregate(table, srcw, dstw, zeros_nh)
    a, b = _tc_post(parts, table, dis, row(b2), row(g2), row(be2),
                    [cW1[:h], cW1[h:2 * h]], False)

    ga, gb = _sc_edge_gather(a, b, src2d, dst2d)
    out = _tc_classifier(ga, gb, edge_attr.T, cW1[2 * h:], row(cb1), cW2,
                         row(cb2), cW3, cb3.reshape(-1, 1))
    return out.T
